# baseline (speedup 1.0000x reference)
import functools

import jax
import jax.numpy as jnp
from jax import lax
from jax.experimental import pallas as pl
from jax.experimental.pallas import tpu as pltpu
from jax.experimental.pallas import tpu_sc as plsc


def _make_sc_gather(n_rows, feat, nw, ch, cw, kbuf):
    perw = ch * cw
    outer = ch // kbuf
    rows = kbuf * cw
    mesh = plsc.VectorSubcoreMesh(core_axis_name="c", subcore_axis_name="s")
    ncores = plsc.get_sparse_core_info().num_cores

    def body(x_hbm, idx_hbm, out_hbm, idx_v, rows0_v, rows1_v, sem_g, sem_w):
        wid = lax.axis_index("s") * ncores + lax.axis_index("c")
        pltpu.sync_copy(idx_hbm.at[wid], idx_v)
        bufs = (rows0_v, rows1_v)

        def step(o, carry):
            def run(buf):
                copies = []
                for k in range(kbuf):
                    copies.append(pltpu.async_copy(
                        x_hbm.at[idx_v.at[o * kbuf + k]],
                        buf.at[pl.ds(k * cw, cw)],
                        sem_g,
                    ))
                for cp in copies:
                    cp.wait()
                dst = out_hbm.at[pl.ds(wid * perw + o * rows, rows)]
                pltpu.async_copy(buf, dst, sem_w)

                @pl.when(o > 0)
                def _():
                    pltpu.make_async_copy(buf, dst, sem_w).wait()

            @pl.when(lax.rem(o, 2) == 0)
            def _():
                run(bufs[0])

            @pl.when(lax.rem(o, 2) == 1)
            def _():
                run(bufs[1])

            return carry

        lax.fori_loop(0, outer, step, 0)
        pltpu.make_async_copy(
            rows0_v, out_hbm.at[pl.ds(wid * perw, rows)], sem_w).wait()

    return pl.kernel(
        body,
        mesh=mesh,
        out_type=jax.ShapeDtypeStruct((nw * perw, feat), jnp.float32),
        scratch_types=[
            pltpu.VMEM((ch, cw), jnp.int32),
            pltpu.VMEM((rows, feat), jnp.float32),
            pltpu.VMEM((rows, feat), jnp.float32),
            pltpu.SemaphoreType.DMA,
            pltpu.SemaphoreType.DMA,
        ],
    )


def _lstm_gates(m_blk, hs, cs, wih_ref, whh_ref, b_ref, hid):
    wdt = wih_ref.dtype
    gates = (jnp.dot(m_blk.astype(wdt), wih_ref[:], preferred_element_type=jnp.float32)
             + jnp.dot(hs.astype(wdt), whh_ref[:], preferred_element_type=jnp.float32)
             + b_ref[:])
    i = jax.nn.sigmoid(gates[:, :hid])
    f = jax.nn.sigmoid(gates[:, hid:2 * hid])
    g = jnp.tanh(gates[:, 2 * hid:3 * hid])
    o = jax.nn.sigmoid(gates[:, 3 * hid:])
    c_new = f * cs + i * g
    h_new = o * jnp.tanh(c_new)
    return c_new, h_new


def _first_body(m_ref, wih_ref, whh_ref, b_ref, out_ref, hs_ref, cs_ref,
                *, steps, nt_rows, hid):
    t = pl.program_id(0)
    r = pl.program_id(1)
    sl = pl.ds(r * nt_rows, nt_rows)

    @pl.when(t == 0)
    def _():
        hs_ref[sl, :] = jnp.zeros((nt_rows, hid), jnp.float32)
        cs_ref[sl, :] = jnp.zeros((nt_rows, hid), jnp.float32)

    c_new, h_new = _lstm_gates(m_ref[0], hs_ref[sl, :], cs_ref[sl, :],
                               wih_ref, whh_ref, b_ref, hid)
    cs_ref[sl, :] = c_new
    hs_ref[sl, :] = h_new

    @pl.when(t == steps - 1)
    def _():
        out_ref[0, sl, :] = h_new
        out_ref[1, sl, :] = c_new


def _layer_body(m_ref, hc_ref, x_ref, wih_ref, whh_ref, b_ref, ws_ref, wn_ref, bo_ref,
                out_ref, hs_ref, cs_ref, *, steps, nt_rows, hid, relu):
    t = pl.program_id(0)
    r = pl.program_id(1)
    sl = pl.ds(r * nt_rows, nt_rows)

    @pl.when(t == 0)
    def _():
        hs_ref[sl, :] = hc_ref[0, sl, :]
        cs_ref[sl, :] = hc_ref[1, sl, :]

    c_new, h_new = _lstm_gates(m_ref[0], hs_ref[sl, :], cs_ref[sl, :],
                               wih_ref, whh_ref, b_ref, hid)
    cs_ref[sl, :] = c_new
    hs_ref[sl, :] = h_new

    @pl.when(t == steps - 1)
    def _():
        rst = (jnp.dot(x_ref[sl, :], ws_ref[:], preferred_element_type=jnp.float32)
               + jnp.dot(h_new, wn_ref[:], preferred_element_type=jnp.float32)
               + bo_ref[:])
        out_ref[sl, :] = jnp.maximum(rst, 0.0) if relu else rst


def _final_body(m_ref, hc_ref, x_ref, wih_ref, whh_ref, b_ref, ws_ref, wn_ref, bo_ref,
                w0_ref, b0_ref, w1_ref, b1_ref, w2_ref, b2_ref,
                wl_ref, bl_ref, out_ref, hs_ref, cs_ref, acch_ref, accx_ref,
                *, steps, nt, nt_rows, hid, n_nodes):
    t = pl.program_id(0)
    r = pl.program_id(1)
    sl = pl.ds(r * nt_rows, nt_rows)

    @pl.when(t == 0)
    def _():
        hs_ref[sl, :] = hc_ref[0, sl, :]
        cs_ref[sl, :] = hc_ref[1, sl, :]

    c_new, h_new = _lstm_gates(m_ref[0], hs_ref[sl, :], cs_ref[sl, :],
                               wih_ref, whh_ref, b_ref, hid)
    cs_ref[sl, :] = c_new
    hs_ref[sl, :] = h_new

    @pl.when(t == steps - 1)
    def _():
        ph = jnp.sum(h_new, axis=0, keepdims=True)
        px = jnp.sum(x_ref[sl, :], axis=0, keepdims=True)

        @pl.when(r == 0)
        def _():
            acch_ref[:, :] = ph
            accx_ref[:, :] = px

        @pl.when(r > 0)
        def _():
            acch_ref[:, :] = acch_ref[:, :] + ph
            accx_ref[:, :] = accx_ref[:, :] + px

        @pl.when(r == nt - 1)
        def _():
            inv_n = jnp.float32(1.0 / n_nodes)
            hm = acch_ref[:, :] * inv_n
            xm = accx_ref[:, :] * inv_n
            rst = (jnp.dot(xm, ws_ref[:], preferred_element_type=jnp.float32)
                   + jnp.dot(hm, wn_ref[:], preferred_element_type=jnp.float32)
                   + bo_ref[:])
            y = jnp.maximum(jnp.dot(rst, w0_ref[:], preferred_element_type=jnp.float32) + b0_ref[:], 0.0)
            y = jnp.maximum(jnp.dot(y, w1_ref[:], preferred_element_type=jnp.float32) + b1_ref[:], 0.0)
            y = jnp.dot(y, w2_ref[:], preferred_element_type=jnp.float32) + b2_ref[:]
            out_ref[:, :] = jnp.dot(y, wl_ref[:], preferred_element_type=jnp.float32) + bl_ref[:]


def _const_spec(shape):
    return pl.BlockSpec(shape, lambda t, r: tuple(0 for _ in shape))


def _cparams():
    return pltpu.CompilerParams(
        dimension_semantics=("arbitrary", "arbitrary"),
        vmem_limit_bytes=100 * 1024 * 1024,
    )


def _m_spec(nt_rows, hid):
    return pl.BlockSpec((1, nt_rows, hid), lambda t, r: (t, r, 0))


def _hc_spec(n, hid):
    return pl.BlockSpec((2, n, hid), lambda t, r: (0, 0, 0))


def _make_first_call(n, steps, hid, nt):
    nt_rows = n // nt
    in_specs = [
        _m_spec(nt_rows, hid),
        _const_spec((hid, 4 * hid)),
        _const_spec((hid, 4 * hid)),
        _const_spec((1, 4 * hid)),
    ]
    return pl.pallas_call(
        functools.partial(_first_body, steps=steps, nt_rows=nt_rows, hid=hid),
        grid=(steps, nt),
        in_specs=in_specs,
        out_specs=_hc_spec(n, hid),
        out_shape=jax.ShapeDtypeStruct((2, n, hid), jnp.float32),
        scratch_shapes=[
            pltpu.VMEM((n, hid), jnp.float32),
            pltpu.VMEM((n, hid), jnp.float32),
        ],
        compiler_params=_cparams(),
    )


def _make_layer_call(n, steps, hid, nt, relu):
    nt_rows = n // nt
    in_specs = [
        _m_spec(nt_rows, hid),
        _hc_spec(n, hid),
        pl.BlockSpec((n, hid), lambda t, r: (0, 0)),
        _const_spec((hid, 4 * hid)),
        _const_spec((hid, 4 * hid)),
        _const_spec((1, 4 * hid)),
        _const_spec((hid, hid)),
        _const_spec((hid, hid)),
        _const_spec((1, hid)),
    ]
    return pl.pallas_call(
        functools.partial(_layer_body, steps=steps, nt_rows=nt_rows, hid=hid, relu=relu),
        grid=(steps, nt),
        in_specs=in_specs,
        out_specs=pl.BlockSpec((n, hid), lambda t, r: (0, 0)),
        out_shape=jax.ShapeDtypeStruct((n, hid), jnp.float32),
        scratch_shapes=[
            pltpu.VMEM((n, hid), jnp.float32),
            pltpu.VMEM((n, hid), jnp.float32),
        ],
        compiler_params=_cparams(),
    )


def _make_final_call(n, steps, hid, nc, nt):
    nt_rows = n // nt
    in_specs = [
        _m_spec(nt_rows, hid),
        _hc_spec(n, hid),
        pl.BlockSpec((n, hid), lambda t, r: (0, 0)),
        _const_spec((hid, 4 * hid)),
        _const_spec((hid, 4 * hid)),
        _const_spec((1, 4 * hid)),
        _const_spec((hid, hid)),
        _const_spec((hid, hid)),
        _const_spec((1, hid)),
        _const_spec((hid, hid)),
        _const_spec((1, hid)),
        _const_spec((hid, hid)),
        _const_spec((1, hid)),
        _const_spec((hid, hid)),
        _const_spec((1, hid)),
        _const_spec((hid, nc)),
        _const_spec((1, nc)),
    ]
    return pl.pallas_call(
        functools.partial(_final_body, steps=steps, nt=nt, nt_rows=nt_rows,
                          hid=hid, n_nodes=n),
        grid=(steps, nt),
        in_specs=in_specs,
        out_specs=pl.BlockSpec((1, nc), lambda t, r: (0, 0)),
        out_shape=jax.ShapeDtypeStruct((1, nc), jnp.float32),
        scratch_shapes=[
            pltpu.VMEM((n, hid), jnp.float32),
            pltpu.VMEM((n, hid), jnp.float32),
            pltpu.VMEM((1, hid), jnp.float32),
            pltpu.VMEM((1, hid), jnp.float32),
        ],
        compiler_params=_cparams(),
    )


def _pick_chunking(perw, row_bytes):
    best = None
    for cw in range(128, 0, -1):
        if perw % cw:
            continue
        ch = perw // cw
        for kbuf in (8, 6, 5, 4, 3, 2, 1):
            if ch % kbuf or (cw * kbuf) % 8:
                continue
            if 2 * cw * kbuf * row_bytes > 420 * 1024:
                continue
            if best is None or (cw * kbuf, kbuf) > (best[0] * best[2], best[2]):
                best = (cw, ch, kbuf)
            break
    return best


def kernel(h, edge_index, conv_params, mlp_params, lin_W, lin_b):
    n, d = h.shape
    e = edge_index.shape[1]
    deg = e // n
    hid = conv_params[0]['W_self'].shape[0]
    nc = lin_W.shape[0]

    info = plsc.get_sparse_core_info()
    nw = info.num_cores * info.num_subcores
    half = deg // 2
    eh = e // 2
    perw = eh // nw
    cw, ch, kbuf = _pick_chunking(perw, hid * 4)

    src = edge_index[0]
    src_sm = jnp.transpose(src.reshape(n, deg)).reshape(2, nw, ch, cw)

    sc_gather = _make_sc_gather(n, hid, nw, ch, cw, kbuf)
    first_call = _make_first_call(n, half, hid, nt=2)
    layer_call = _make_layer_call(n, half, hid, nt=2, relu=True)
    final_call = _make_final_call(n, half, hid, nc, nt=2)

    def layer_weights(p):
        bias = (p['b_ih'] + p['b_hh']).reshape(1, -1)
        return (jnp.transpose(p['W_ih']).astype(jnp.bfloat16),
                jnp.transpose(p['W_hh']).astype(jnp.bfloat16), bias,
                jnp.transpose(p['W_self']),
                jnp.transpose(p['W_neigh']), p['b'].reshape(1, -1))

    mlp = []
    for p in mlp_params:
        mlp.extend([jnp.transpose(p['W']), p['b'].reshape(1, -1)])

    x = h
    for li in range(6):
        wih, whh, bias, ws, wn, bo = layer_weights(conv_params[li])
        m_a = sc_gather(x, src_sm[0]).reshape(half, n, hid)
        m_b = sc_gather(x, src_sm[1]).reshape(half, n, hid)
        hc = first_call(m_a, wih, whh, bias)
        if li < 5:
            x = layer_call(m_b, hc, x, wih, whh, bias, ws, wn, bo)
        else:
            return final_call(m_b, hc, x, wih, whh, bias, ws, wn, bo, *mlp,
                              jnp.transpose(lin_W), lin_b.reshape(1, -1))


# speedup vs baseline: 1.3316x; 1.3316x over previous
import functools

import jax
import jax.numpy as jnp
from jax import lax
from jax.experimental import pallas as pl
from jax.experimental.pallas import tpu as pltpu
from jax.experimental.pallas import tpu_sc as plsc


def _make_sc_gather(n_rows, feat, nw, ch, cw, kbuf):
    perw = ch * cw
    outer = ch // kbuf
    rows = kbuf * cw
    mesh = plsc.VectorSubcoreMesh(core_axis_name="c", subcore_axis_name="s")
    ncores = plsc.get_sparse_core_info().num_cores

    def body(x_hbm, idx_hbm, out_hbm, idx_v, rows0_v, rows1_v, sem_g, sem_w):
        wid = lax.axis_index("s") * ncores + lax.axis_index("c")
        pltpu.sync_copy(idx_hbm.at[wid], idx_v)
        bufs = (rows0_v, rows1_v)

        def step(o, carry):
            def run(buf):
                copies = []
                for k in range(kbuf):
                    copies.append(pltpu.async_copy(
                        x_hbm.at[idx_v.at[o * kbuf + k]],
                        buf.at[pl.ds(k * cw, cw)],
                        sem_g,
                    ))
                for cp in copies:
                    cp.wait()
                dst = out_hbm.at[pl.ds(wid * perw + o * rows, rows)]
                pltpu.async_copy(buf, dst, sem_w)

                @pl.when(o > 0)
                def _():
                    pltpu.make_async_copy(buf, dst, sem_w).wait()

            @pl.when(lax.rem(o, 2) == 0)
            def _():
                run(bufs[0])

            @pl.when(lax.rem(o, 2) == 1)
            def _():
                run(bufs[1])

            return carry

        lax.fori_loop(0, outer, step, 0)
        pltpu.make_async_copy(
            rows0_v, out_hbm.at[pl.ds(wid * perw, rows)], sem_w).wait()

    return pl.kernel(
        body,
        mesh=mesh,
        out_type=jax.ShapeDtypeStruct((nw * perw, feat), jnp.float32),
        scratch_types=[
            pltpu.VMEM((ch, cw), jnp.int32),
            pltpu.VMEM((rows, feat), jnp.float32),
            pltpu.VMEM((rows, feat), jnp.float32),
            pltpu.SemaphoreType.DMA,
            pltpu.SemaphoreType.DMA,
        ],
    )


def _lstm_gates(m_blk, hs, cs, wcat_ref, b_ref, hid):
    wdt = wcat_ref.dtype
    xx = jnp.concatenate([m_blk.astype(wdt), hs.astype(wdt)], axis=-1)
    gates = jnp.dot(xx, wcat_ref[:], preferred_element_type=jnp.float32) + b_ref[:]
    i = jax.nn.sigmoid(gates[:, :hid])
    f = jax.nn.sigmoid(gates[:, hid:2 * hid])
    g = jnp.tanh(gates[:, 2 * hid:3 * hid])
    o = jax.nn.sigmoid(gates[:, 3 * hid:])
    c_new = f * cs + i * g
    h_new = o * jnp.tanh(c_new)
    return c_new, h_new


def _first_body(m_ref, wcat_ref, b_ref, out_ref, hs_ref, cs_ref,
                *, steps, nt_rows, hid):
    t = pl.program_id(0)
    r = pl.program_id(1)
    sl = pl.ds(r * nt_rows, nt_rows)

    @pl.when(t == 0)
    def _():
        hs_ref[sl, :] = jnp.zeros((nt_rows, hid), jnp.float32)
        cs_ref[sl, :] = jnp.zeros((nt_rows, hid), jnp.float32)

    c_new, h_new = _lstm_gates(m_ref[0], hs_ref[sl, :], cs_ref[sl, :],
                               wcat_ref, b_ref, hid)
    cs_ref[sl, :] = c_new
    hs_ref[sl, :] = h_new

    @pl.when(t == steps - 1)
    def _():
        out_ref[0, sl, :] = h_new
        out_ref[1, sl, :] = c_new


def _layer_body(m_ref, hc_ref, x_ref, wcat_ref, b_ref, ws_ref, wn_ref, bo_ref,
                out_ref, hs_ref, cs_ref, *, steps, nt_rows, hid, relu):
    t = pl.program_id(0)
    r = pl.program_id(1)
    sl = pl.ds(r * nt_rows, nt_rows)

    @pl.when(t == 0)
    def _():
        hs_ref[sl, :] = hc_ref[0, sl, :]
        cs_ref[sl, :] = hc_ref[1, sl, :]

    c_new, h_new = _lstm_gates(m_ref[0], hs_ref[sl, :], cs_ref[sl, :],
                               wcat_ref, b_ref, hid)
    cs_ref[sl, :] = c_new
    hs_ref[sl, :] = h_new

    @pl.when(t == steps - 1)
    def _():
        rst = (jnp.dot(x_ref[sl, :], ws_ref[:], preferred_element_type=jnp.float32)
               + jnp.dot(h_new, wn_ref[:], preferred_element_type=jnp.float32)
               + bo_ref[:])
        out_ref[sl, :] = jnp.maximum(rst, 0.0) if relu else rst


def _final_body(m_ref, hc_ref, x_ref, wcat_ref, b_ref, ws_ref, wn_ref, bo_ref,
                w0_ref, b0_ref, w1_ref, b1_ref, w2_ref, b2_ref,
                wl_ref, bl_ref, out_ref, hs_ref, cs_ref, acch_ref, accx_ref,
                *, steps, nt, nt_rows, hid, n_nodes):
    t = pl.program_id(0)
    r = pl.program_id(1)
    sl = pl.ds(r * nt_rows, nt_rows)

    @pl.when(t == 0)
    def _():
        hs_ref[sl, :] = hc_ref[0, sl, :]
        cs_ref[sl, :] = hc_ref[1, sl, :]

    c_new, h_new = _lstm_gates(m_ref[0], hs_ref[sl, :], cs_ref[sl, :],
                               wcat_ref, b_ref, hid)
    cs_ref[sl, :] = c_new
    hs_ref[sl, :] = h_new

    @pl.when(t == steps - 1)
    def _():
        ph = jnp.sum(h_new, axis=0, keepdims=True)
        px = jnp.sum(x_ref[sl, :], axis=0, keepdims=True)

        @pl.when(r == 0)
        def _():
            acch_ref[:, :] = ph
            accx_ref[:, :] = px

        @pl.when(r > 0)
        def _():
            acch_ref[:, :] = acch_ref[:, :] + ph
            accx_ref[:, :] = accx_ref[:, :] + px

        @pl.when(r == nt - 1)
        def _():
            inv_n = jnp.float32(1.0 / n_nodes)
            hm = acch_ref[:, :] * inv_n
            xm = accx_ref[:, :] * inv_n
            rst = (jnp.dot(xm, ws_ref[:], preferred_element_type=jnp.float32)
                   + jnp.dot(hm, wn_ref[:], preferred_element_type=jnp.float32)
                   + bo_ref[:])
            y = jnp.maximum(jnp.dot(rst, w0_ref[:], preferred_element_type=jnp.float32) + b0_ref[:], 0.0)
            y = jnp.maximum(jnp.dot(y, w1_ref[:], preferred_element_type=jnp.float32) + b1_ref[:], 0.0)
            y = jnp.dot(y, w2_ref[:], preferred_element_type=jnp.float32) + b2_ref[:]
            out_ref[:, :] = jnp.dot(y, wl_ref[:], preferred_element_type=jnp.float32) + bl_ref[:]


def _const_spec(shape):
    return pl.BlockSpec(shape, lambda t, r: tuple(0 for _ in shape))


def _cparams():
    return pltpu.CompilerParams(
        dimension_semantics=("arbitrary", "arbitrary"),
        vmem_limit_bytes=100 * 1024 * 1024,
    )


def _m_spec(nt_rows, hid):
    return pl.BlockSpec((1, nt_rows, hid), lambda t, r: (t, r, 0))


def _hc_spec(n, hid):
    return pl.BlockSpec((2, n, hid), lambda t, r: (0, 0, 0))


def _make_first_call(n, steps, hid, nt):
    nt_rows = n // nt
    in_specs = [
        _m_spec(nt_rows, hid),
        _const_spec((2 * hid, 4 * hid)),
        _const_spec((1, 4 * hid)),
    ]
    return pl.pallas_call(
        functools.partial(_first_body, steps=steps, nt_rows=nt_rows, hid=hid),
        grid=(steps, nt),
        in_specs=in_specs,
        out_specs=_hc_spec(n, hid),
        out_shape=jax.ShapeDtypeStruct((2, n, hid), jnp.float32),
        scratch_shapes=[
            pltpu.VMEM((n, hid), jnp.float32),
            pltpu.VMEM((n, hid), jnp.float32),
        ],
        compiler_params=_cparams(),
    )


def _make_layer_call(n, steps, hid, nt, relu):
    nt_rows = n // nt
    in_specs = [
        _m_spec(nt_rows, hid),
        _hc_spec(n, hid),
        pl.BlockSpec((n, hid), lambda t, r: (0, 0)),
        _const_spec((2 * hid, 4 * hid)),
        _const_spec((1, 4 * hid)),
        _const_spec((hid, hid)),
        _const_spec((hid, hid)),
        _const_spec((1, hid)),
    ]
    return pl.pallas_call(
        functools.partial(_layer_body, steps=steps, nt_rows=nt_rows, hid=hid, relu=relu),
        grid=(steps, nt),
        in_specs=in_specs,
        out_specs=pl.BlockSpec((n, hid), lambda t, r: (0, 0)),
        out_shape=jax.ShapeDtypeStruct((n, hid), jnp.float32),
        scratch_shapes=[
            pltpu.VMEM((n, hid), jnp.float32),
            pltpu.VMEM((n, hid), jnp.float32),
        ],
        compiler_params=_cparams(),
    )


def _make_final_call(n, steps, hid, nc, nt):
    nt_rows = n // nt
    in_specs = [
        _m_spec(nt_rows, hid),
        _hc_spec(n, hid),
        pl.BlockSpec((n, hid), lambda t, r: (0, 0)),
        _const_spec((2 * hid, 4 * hid)),
        _const_spec((1, 4 * hid)),
        _const_spec((hid, hid)),
        _const_spec((hid, hid)),
        _const_spec((1, hid)),
        _const_spec((hid, hid)),
        _const_spec((1, hid)),
        _const_spec((hid, hid)),
        _const_spec((1, hid)),
        _const_spec((hid, hid)),
        _const_spec((1, hid)),
        _const_spec((hid, nc)),
        _const_spec((1, nc)),
    ]
    return pl.pallas_call(
        functools.partial(_final_body, steps=steps, nt=nt, nt_rows=nt_rows,
                          hid=hid, n_nodes=n),
        grid=(steps, nt),
        in_specs=in_specs,
        out_specs=pl.BlockSpec((1, nc), lambda t, r: (0, 0)),
        out_shape=jax.ShapeDtypeStruct((1, nc), jnp.float32),
        scratch_shapes=[
            pltpu.VMEM((n, hid), jnp.float32),
            pltpu.VMEM((n, hid), jnp.float32),
            pltpu.VMEM((1, hid), jnp.float32),
            pltpu.VMEM((1, hid), jnp.float32),
        ],
        compiler_params=_cparams(),
    )


def _pick_chunking(perw, row_bytes):
    best = None
    for cw in range(128, 0, -1):
        if perw % cw:
            continue
        ch = perw // cw
        for kbuf in (8, 6, 5, 4, 3, 2, 1):
            if ch % kbuf or (cw * kbuf) % 8:
                continue
            if 2 * cw * kbuf * row_bytes > 420 * 1024:
                continue
            if best is None or (cw * kbuf, kbuf) > (best[0] * best[2], best[2]):
                best = (cw, ch, kbuf)
            break
    return best


def kernel(h, edge_index, conv_params, mlp_params, lin_W, lin_b):
    n, d = h.shape
    e = edge_index.shape[1]
    deg = e // n
    hid = conv_params[0]['W_self'].shape[0]
    nc = lin_W.shape[0]

    info = plsc.get_sparse_core_info()
    nw = info.num_cores * info.num_subcores
    half = deg // 2
    eh = e // 2
    perw = eh // nw
    cw, ch, kbuf = _pick_chunking(perw, hid * 4)

    src = edge_index[0]
    src_sm = jnp.transpose(src.reshape(n, deg)).reshape(2, nw, ch, cw)

    sc_gather = _make_sc_gather(n, hid, nw, ch, cw, kbuf)
    first_call = _make_first_call(n, half, hid, nt=1)
    layer_call = _make_layer_call(n, half, hid, nt=1, relu=True)
    final_call = _make_final_call(n, half, hid, nc, nt=1)

    def layer_weights(p):
        wcat = jnp.transpose(jnp.concatenate([p['W_ih'], p['W_hh']], axis=1))
        bias = (p['b_ih'] + p['b_hh']).reshape(1, -1)
        return (wcat.astype(jnp.bfloat16), bias, jnp.transpose(p['W_self']),
                jnp.transpose(p['W_neigh']), p['b'].reshape(1, -1))

    mlp = []
    for p in mlp_params:
        mlp.extend([jnp.transpose(p['W']), p['b'].reshape(1, -1)])

    x = h
    for li in range(6):
        wcat, bias, ws, wn, bo = layer_weights(conv_params[li])
        m_a = sc_gather(x, src_sm[0]).reshape(half, n, hid)
        m_b = sc_gather(x, src_sm[1]).reshape(half, n, hid)
        hc = first_call(m_a, wcat, bias)
        if li < 5:
            x = layer_call(m_b, hc, x, wcat, bias, ws, wn, bo)
        else:
            return final_call(m_b, hc, x, wcat, bias, ws, wn, bo, *mlp,
                              jnp.transpose(lin_W), lin_b.reshape(1, -1))


# speedup vs baseline: 1.3392x; 1.0057x over previous
import functools

import jax
import jax.numpy as jnp
from jax import lax
from jax.experimental import pallas as pl
from jax.experimental.pallas import tpu as pltpu
from jax.experimental.pallas import tpu_sc as plsc


def _make_sc_gather(n_rows, feat, nw, ch, cw, kbuf):
    perw = ch * cw
    outer = ch // kbuf
    rows = kbuf * cw
    mesh = plsc.VectorSubcoreMesh(core_axis_name="c", subcore_axis_name="s")
    ncores = plsc.get_sparse_core_info().num_cores

    def body(x_hbm, idx_hbm, out_hbm, idx_v, rows0_v, rows1_v, sem_g, sem_w):
        wid = lax.axis_index("s") * ncores + lax.axis_index("c")
        pltpu.sync_copy(idx_hbm.at[wid], idx_v)
        bufs = (rows0_v, rows1_v)

        def step(o, carry):
            def run(buf):
                copies = []
                for k in range(kbuf):
                    copies.append(pltpu.async_copy(
                        x_hbm.at[idx_v.at[o * kbuf + k]],
                        buf.at[pl.ds(k * cw, cw)],
                        sem_g,
                    ))
                for cp in copies:
                    cp.wait()
                dst = out_hbm.at[pl.ds(wid * perw + o * rows, rows)]
                pltpu.async_copy(buf, dst, sem_w)

                @pl.when(o > 0)
                def _():
                    pltpu.make_async_copy(buf, dst, sem_w).wait()

            @pl.when(lax.rem(o, 2) == 0)
            def _():
                run(bufs[0])

            @pl.when(lax.rem(o, 2) == 1)
            def _():
                run(bufs[1])

            return carry

        lax.fori_loop(0, outer, step, 0)
        pltpu.make_async_copy(
            rows0_v, out_hbm.at[pl.ds(wid * perw, rows)], sem_w).wait()

    return pl.kernel(
        body,
        mesh=mesh,
        out_type=jax.ShapeDtypeStruct((nw * perw, feat), jnp.float32),
        scratch_types=[
            pltpu.VMEM((ch, cw), jnp.int32),
            pltpu.VMEM((rows, feat), jnp.float32),
            pltpu.VMEM((rows, feat), jnp.float32),
            pltpu.SemaphoreType.DMA,
            pltpu.SemaphoreType.DMA,
        ],
    )


def _lstm_gates(m_blk, hs, cs, wcat_ref, b_ref, hid):
    wdt = wcat_ref.dtype
    xx = jnp.concatenate([m_blk.astype(wdt), hs.astype(wdt)], axis=-1)
    gates = jnp.dot(xx, wcat_ref[:], preferred_element_type=jnp.float32) + b_ref[:]
    i = jax.nn.sigmoid(gates[:, :hid])
    f = jax.nn.sigmoid(gates[:, hid:2 * hid])
    g = jnp.tanh(gates[:, 2 * hid:3 * hid])
    o = jax.nn.sigmoid(gates[:, 3 * hid:])
    c_new = f * cs + i * g
    h_new = o * jnp.tanh(c_new)
    return c_new, h_new


def _first_body(m_ref, wcat_ref, b_ref, out_ref, hs_ref, cs_ref,
                *, steps, nt_rows, hid):
    t = pl.program_id(0)
    r = pl.program_id(1)
    sl = pl.ds(r * nt_rows, nt_rows)

    @pl.when(t == 0)
    def _():
        hs_ref[sl, :] = jnp.zeros((nt_rows, hid), jnp.float32)
        cs_ref[sl, :] = jnp.zeros((nt_rows, hid), jnp.float32)

    c_new, h_new = _lstm_gates(m_ref[0], hs_ref[sl, :], cs_ref[sl, :],
                               wcat_ref, b_ref, hid)
    cs_ref[sl, :] = c_new
    hs_ref[sl, :] = h_new

    @pl.when(t == steps - 1)
    def _():
        out_ref[0, sl, :] = h_new
        out_ref[1, sl, :] = c_new


def _layer_body(m_ref, hc_ref, x_ref, wcat_ref, b_ref, ws_ref, wn_ref, bo_ref,
                out_ref, hs_ref, cs_ref, *, steps, nt_rows, hid, relu):
    t = pl.program_id(0)
    r = pl.program_id(1)
    sl = pl.ds(r * nt_rows, nt_rows)

    @pl.when(t == 0)
    def _():
        hs_ref[sl, :] = hc_ref[0, sl, :]
        cs_ref[sl, :] = hc_ref[1, sl, :]

    c_new, h_new = _lstm_gates(m_ref[0], hs_ref[sl, :], cs_ref[sl, :],
                               wcat_ref, b_ref, hid)
    cs_ref[sl, :] = c_new
    hs_ref[sl, :] = h_new

    @pl.when(t == steps - 1)
    def _():
        rst = (jnp.dot(x_ref[sl, :], ws_ref[:], preferred_element_type=jnp.float32)
               + jnp.dot(h_new, wn_ref[:], preferred_element_type=jnp.float32)
               + bo_ref[:])
        out_ref[sl, :] = jnp.maximum(rst, 0.0) if relu else rst


def _final_body(m_ref, hc_ref, x_ref, wcat_ref, b_ref, ws_ref, wn_ref, bo_ref,
                w0_ref, b0_ref, w1_ref, b1_ref, w2_ref, b2_ref,
                wl_ref, bl_ref, out_ref, hs_ref, cs_ref, acch_ref, accx_ref,
                *, steps, nt, nt_rows, hid, n_nodes):
    t = pl.program_id(0)
    r = pl.program_id(1)
    sl = pl.ds(r * nt_rows, nt_rows)

    @pl.when(t == 0)
    def _():
        hs_ref[sl, :] = hc_ref[0, sl, :]
        cs_ref[sl, :] = hc_ref[1, sl, :]

    c_new, h_new = _lstm_gates(m_ref[0], hs_ref[sl, :], cs_ref[sl, :],
                               wcat_ref, b_ref, hid)
    cs_ref[sl, :] = c_new
    hs_ref[sl, :] = h_new

    @pl.when(t == steps - 1)
    def _():
        ph = jnp.sum(h_new, axis=0, keepdims=True)
        px = jnp.sum(x_ref[sl, :], axis=0, keepdims=True)

        @pl.when(r == 0)
        def _():
            acch_ref[:, :] = ph
            accx_ref[:, :] = px

        @pl.when(r > 0)
        def _():
            acch_ref[:, :] = acch_ref[:, :] + ph
            accx_ref[:, :] = accx_ref[:, :] + px

        @pl.when(r == nt - 1)
        def _():
            inv_n = jnp.float32(1.0 / n_nodes)
            hm = acch_ref[:, :] * inv_n
            xm = accx_ref[:, :] * inv_n
            rst = (jnp.dot(xm, ws_ref[:], preferred_element_type=jnp.float32)
                   + jnp.dot(hm, wn_ref[:], preferred_element_type=jnp.float32)
                   + bo_ref[:])
            y = jnp.maximum(jnp.dot(rst, w0_ref[:], preferred_element_type=jnp.float32) + b0_ref[:], 0.0)
            y = jnp.maximum(jnp.dot(y, w1_ref[:], preferred_element_type=jnp.float32) + b1_ref[:], 0.0)
            y = jnp.dot(y, w2_ref[:], preferred_element_type=jnp.float32) + b2_ref[:]
            out_ref[:, :] = jnp.dot(y, wl_ref[:], preferred_element_type=jnp.float32) + bl_ref[:]


def _const_spec(shape):
    return pl.BlockSpec(shape, lambda t, r: tuple(0 for _ in shape))


def _cparams():
    return pltpu.CompilerParams(
        dimension_semantics=("arbitrary", "arbitrary"),
        vmem_limit_bytes=100 * 1024 * 1024,
    )


def _m_spec(nt_rows, hid):
    return pl.BlockSpec((1, nt_rows, hid), lambda t, r: (t, r, 0))


def _hc_spec(n, hid):
    return pl.BlockSpec((2, n, hid), lambda t, r: (0, 0, 0))


def _make_first_call(n, steps, hid, nt):
    nt_rows = n // nt
    in_specs = [
        _m_spec(nt_rows, hid),
        _const_spec((2 * hid, 4 * hid)),
        _const_spec((1, 4 * hid)),
    ]
    return pl.pallas_call(
        functools.partial(_first_body, steps=steps, nt_rows=nt_rows, hid=hid),
        grid=(steps, nt),
        in_specs=in_specs,
        out_specs=_hc_spec(n, hid),
        out_shape=jax.ShapeDtypeStruct((2, n, hid), jnp.float32),
        scratch_shapes=[
            pltpu.VMEM((n, hid), jnp.float32),
            pltpu.VMEM((n, hid), jnp.float32),
        ],
        compiler_params=_cparams(),
    )


def _make_layer_call(n, steps, hid, nt, relu):
    nt_rows = n // nt
    in_specs = [
        _m_spec(nt_rows, hid),
        _hc_spec(n, hid),
        pl.BlockSpec((n, hid), lambda t, r: (0, 0)),
        _const_spec((2 * hid, 4 * hid)),
        _const_spec((1, 4 * hid)),
        _const_spec((hid, hid)),
        _const_spec((hid, hid)),
        _const_spec((1, hid)),
    ]
    return pl.pallas_call(
        functools.partial(_layer_body, steps=steps, nt_rows=nt_rows, hid=hid, relu=relu),
        grid=(steps, nt),
        in_specs=in_specs,
        out_specs=pl.BlockSpec((n, hid), lambda t, r: (0, 0)),
        out_shape=jax.ShapeDtypeStruct((n, hid), jnp.float32),
        scratch_shapes=[
            pltpu.VMEM((n, hid), jnp.float32),
            pltpu.VMEM((n, hid), jnp.float32),
        ],
        compiler_params=_cparams(),
    )


def _make_final_call(n, steps, hid, nc, nt):
    nt_rows = n // nt
    in_specs = [
        _m_spec(nt_rows, hid),
        _hc_spec(n, hid),
        pl.BlockSpec((n, hid), lambda t, r: (0, 0)),
        _const_spec((2 * hid, 4 * hid)),
        _const_spec((1, 4 * hid)),
        _const_spec((hid, hid)),
        _const_spec((hid, hid)),
        _const_spec((1, hid)),
        _const_spec((hid, hid)),
        _const_spec((1, hid)),
        _const_spec((hid, hid)),
        _const_spec((1, hid)),
        _const_spec((hid, hid)),
        _const_spec((1, hid)),
        _const_spec((hid, nc)),
        _const_spec((1, nc)),
    ]
    return pl.pallas_call(
        functools.partial(_final_body, steps=steps, nt=nt, nt_rows=nt_rows,
                          hid=hid, n_nodes=n),
        grid=(steps, nt),
        in_specs=in_specs,
        out_specs=pl.BlockSpec((1, nc), lambda t, r: (0, 0)),
        out_shape=jax.ShapeDtypeStruct((1, nc), jnp.float32),
        scratch_shapes=[
            pltpu.VMEM((n, hid), jnp.float32),
            pltpu.VMEM((n, hid), jnp.float32),
            pltpu.VMEM((1, hid), jnp.float32),
            pltpu.VMEM((1, hid), jnp.float32),
        ],
        compiler_params=_cparams(),
    )


def _pick_chunking(perw, row_bytes):
    best = None
    for cw in range(128, 0, -1):
        if perw % cw:
            continue
        ch = perw // cw
        for kbuf in (8, 6, 5, 4, 3, 2, 1):
            if ch % kbuf or (cw * kbuf) % 8:
                continue
            if 2 * cw * kbuf * row_bytes > 420 * 1024:
                continue
            if best is None or (cw * kbuf, kbuf) > (best[0] * best[2], best[2]):
                best = (cw, ch, kbuf)
            break
    return best


def kernel(h, edge_index, conv_params, mlp_params, lin_W, lin_b):
    n, d = h.shape
    e = edge_index.shape[1]
    deg = e // n
    hid = conv_params[0]['W_self'].shape[0]
    nc = lin_W.shape[0]

    info = plsc.get_sparse_core_info()
    nw = info.num_cores * info.num_subcores
    half = deg // 2
    eh = e // 2
    perw = eh // nw
    cw, ch, kbuf = _pick_chunking(perw, hid * 4)

    src = edge_index[0]
    src_sm = jnp.transpose(src.reshape(n, deg)).reshape(2, nw, ch, cw)

    sc_gather = _make_sc_gather(n, hid, nw, ch, cw, kbuf)
    first_call = _make_first_call(n, half, hid, nt=1)
    layer_call = _make_layer_call(n, half, hid, nt=1, relu=True)
    final_call = _make_final_call(n, half, hid, nc, nt=1)

    def layer_weights(p):
        wcat = jnp.transpose(jnp.concatenate([p['W_ih'], p['W_hh']], axis=1))
        bias = (p['b_ih'] + p['b_hh']).reshape(1, -1)
        return (wcat, bias, jnp.transpose(p['W_self']),
                jnp.transpose(p['W_neigh']), p['b'].reshape(1, -1))

    mlp = []
    for p in mlp_params:
        mlp.extend([jnp.transpose(p['W']), p['b'].reshape(1, -1)])

    x = h
    for li in range(6):
        wcat, bias, ws, wn, bo = layer_weights(conv_params[li])
        m_a = sc_gather(x, src_sm[0]).reshape(half, n, hid)
        m_b = sc_gather(x, src_sm[1]).reshape(half, n, hid)
        hc = first_call(m_a, wcat, bias)
        if li < 5:
            x = layer_call(m_b, hc, x, wcat, bias, ws, wn, bo)
        else:
            return final_call(m_b, hc, x, wcat, bias, ws, wn, bo, *mlp,
                              jnp.transpose(lin_W), lin_b.reshape(1, -1))
